# Initial kernel scaffold; baseline (speedup 1.0000x reference)
import jax
import jax.numpy as jnp
from jax.experimental import pallas as pl


def kernel(atom_fea, nbr_fea, nbr_fea_idx, crystal_atom_idx, params):
    raise NotImplementedError("write your pallas kernel here")


# speedup vs baseline: 1.8973x; 1.8973x over previous
import functools

import jax
import jax.numpy as jnp
from jax import lax
from jax.experimental import pallas as pl
from jax.experimental.pallas import tpu as pltpu
from jax.experimental.pallas import tpu_sc as plsc

N = 50000
M = 16
ORIG = 92
NBR = 16
ATOM = 64
F2 = 2 * ATOM
H = 128
N0 = 1000
APC = 50
E = N * M
EPS = 1e-5

NW = 32
EPW = E // NW
CHUNK = 128
NFULL = EPW // CHUNK
TAIL = EPW - NFULL * CHUNK
NBUF = 2


def _sc_gather_body(table_hbm, idx_hbm, out_hbm, idx_v, rows_v, *sems):
    gsems = sems[:NBUF]
    wsems = sems[NBUF:2 * NBUF]
    isems = sems[2 * NBUF:]
    wid = lax.axis_index("s") * 2 + lax.axis_index("c")
    base = wid * EPW

    def fire_idx(g, b, n=CHUNK):
        pltpu.async_copy(
            idx_hbm.at[pl.ds(base + g * CHUNK, n)],
            idx_v.at[b].at[pl.ds(0, n)], isems[b])

    def wait_idx(b, n=CHUNK):
        pltpu.make_async_copy(
            idx_hbm.at[pl.ds(base, n)],
            idx_v.at[b].at[pl.ds(0, n)], isems[b]).wait()

    def fire_gather(b, n=CHUNK):
        pltpu.async_copy(
            table_hbm.at[idx_v.at[b].at[pl.ds(0, n)]],
            rows_v.at[b].at[pl.ds(0, n)], gsems[b])

    def drain_gather(b, n=CHUNK):
        pltpu.make_async_copy(
            rows_v.at[b].at[pl.ds(0, n)],
            out_hbm.at[pl.ds(base, n)], gsems[b]).wait()

    def fire_write(g, b, n=CHUNK):
        pltpu.async_copy(
            rows_v.at[b].at[pl.ds(0, n)],
            out_hbm.at[pl.ds(base + g * CHUNK, n)], wsems[b])

    def wait_write(b, n=CHUNK):
        pltpu.make_async_copy(
            rows_v.at[b].at[pl.ds(0, n)],
            out_hbm.at[pl.ds(base, n)], wsems[b]).wait()

    fire_idx(0, 0)
    fire_idx(1, 1)
    wait_idx(0)
    fire_gather(0)
    wait_idx(1)
    fire_gather(1)

    def body(i, carry):
        for b in range(NBUF):
            g = i * NBUF + b
            drain_gather(b)

            @pl.when(g + NBUF < NFULL)
            def _():
                fire_idx(g + NBUF, b)

            fire_write(g, b)

            @pl.when(g + NBUF < NFULL)
            def _():
                wait_write(b)
                wait_idx(b)
                fire_gather(b)
        return carry

    lax.fori_loop(0, (NFULL - 1) // NBUF, body, None)
    drain_gather(0)
    fire_write(NFULL - 1, 0)
    wait_write(1)
    fire_idx(NFULL, 1, TAIL)
    wait_idx(1, TAIL)
    fire_gather(1, TAIL)
    drain_gather(1, TAIL)
    fire_write(NFULL, 1, TAIL)
    wait_write(1, TAIL)
    wait_write(0)


def _sc_gather(table, idx_flat):
    mesh = plsc.VectorSubcoreMesh(core_axis_name="c", subcore_axis_name="s")
    return pl.kernel(
        _sc_gather_body,
        out_type=jax.ShapeDtypeStruct((E, F2), jnp.float32),
        mesh=mesh,
        scratch_types=(
            [pltpu.VMEM((NBUF, CHUNK), jnp.int32),
             pltpu.VMEM((NBUF, CHUNK, F2), jnp.float32)]
            + [pltpu.SemaphoreType.DMA] * (3 * NBUF)
        ),
    )(table, idx_flat)


def _embed_body(af_ref, mask_ref, w_ref, b_ref, am_ref, x_ref):
    am = af_ref[...] * mask_ref[...]
    am_ref[...] = am
    x_ref[...] = jnp.dot(am, w_ref[...],
                         preferred_element_type=jnp.float32,
                         precision=lax.Precision.HIGHEST) + b_ref[...]


def _embed(atom_fea, mask, emb_W, emb_b):
    RB = 2000
    grid = N // RB
    return pl.pallas_call(
        _embed_body,
        grid=(grid,),
        in_specs=[
            pl.BlockSpec((RB, ORIG), lambda i: (i, 0)),
            pl.BlockSpec((1, ORIG), lambda i: (0, 0)),
            pl.BlockSpec((ORIG, ATOM), lambda i: (0, 0)),
            pl.BlockSpec((1, ATOM), lambda i: (0, 0)),
        ],
        out_specs=[
            pl.BlockSpec((RB, ORIG), lambda i: (i, 0)),
            pl.BlockSpec((RB, ATOM), lambda i: (i, 0)),
        ],
        out_shape=[
            jax.ShapeDtypeStruct((N, ORIG), jnp.float32),
            jax.ShapeDtypeStruct((N, ATOM), jnp.float32),
        ],
    )(atom_fea, mask, emb_W, emb_b)


def _dense_body(x_ref, ws_ref, wn_ref, b_ref, ys_ref, yn_ref):
    x = x_ref[...]
    ys_ref[...] = jnp.dot(x, ws_ref[...],
                          preferred_element_type=jnp.float32,
                          precision=lax.Precision.HIGHEST) + b_ref[...]
    yn_ref[...] = jnp.dot(x, wn_ref[...],
                          preferred_element_type=jnp.float32,
                          precision=lax.Precision.HIGHEST)


def _dense(x, Ws, Wn, b):
    RB = 2000
    return pl.pallas_call(
        _dense_body,
        grid=(N // RB,),
        in_specs=[
            pl.BlockSpec((RB, ATOM), lambda i: (i, 0)),
            pl.BlockSpec((ATOM, F2), lambda i: (0, 0)),
            pl.BlockSpec((ATOM, F2), lambda i: (0, 0)),
            pl.BlockSpec((1, F2), lambda i: (0, 0)),
        ],
        out_specs=[
            pl.BlockSpec((RB, F2), lambda i: (i, 0)),
            pl.BlockSpec((RB, F2), lambda i: (i, 0)),
        ],
        out_shape=[
            jax.ShapeDtypeStruct((N, F2), jnp.float32),
            jax.ShapeDtypeStruct((N, F2), jnp.float32),
        ],
    )(x, Ws, Wn, b)


AB = 400
EB = AB * M
EGRID = N // AB


def _stats_body(edges_ref, nbr_ref, ys_ref, we_ref, sum_ref, sq_ref):
    v = jnp.dot(nbr_ref[...], we_ref[...],
                preferred_element_type=jnp.float32,
                precision=lax.Precision.HIGHEST)
    g = edges_ref[...] + v
    g = g.reshape(AB, M, F2) + ys_ref[...][:, None, :]

    @pl.when(pl.program_id(0) == 0)
    def _():
        sum_ref[...] = jnp.zeros_like(sum_ref)
        sq_ref[...] = jnp.zeros_like(sq_ref)

    sum_ref[...] += jnp.sum(g, axis=(0, 1))[None, :]
    sq_ref[...] += jnp.sum(g * g, axis=(0, 1))[None, :]


def _stats(edges, nbr2, ys, We):
    return pl.pallas_call(
        _stats_body,
        grid=(EGRID,),
        in_specs=[
            pl.BlockSpec((EB, F2), lambda i: (i, 0)),
            pl.BlockSpec((EB, NBR), lambda i: (i, 0)),
            pl.BlockSpec((AB, F2), lambda i: (i, 0)),
            pl.BlockSpec((NBR, F2), lambda i: (0, 0)),
        ],
        out_specs=[
            pl.BlockSpec((1, F2), lambda i: (0, 0)),
            pl.BlockSpec((1, F2), lambda i: (0, 0)),
        ],
        out_shape=[
            jax.ShapeDtypeStruct((1, F2), jnp.float32),
            jax.ShapeDtypeStruct((1, F2), jnp.float32),
        ],
    )(edges, nbr2, ys, We)


def _gate_body(edges_ref, nbr_ref, ys_ref, we_ref, sc_ref, sh_ref,
               s_ref, sum_ref, sq_ref):
    v = jnp.dot(nbr_ref[...], we_ref[...],
                preferred_element_type=jnp.float32,
                precision=lax.Precision.HIGHEST)
    g = edges_ref[...] + v
    g = g.reshape(AB, M, F2) + ys_ref[...][:, None, :]
    g = g * sc_ref[...][None, :, :] + sh_ref[...][None, :, :]
    filt = jax.nn.sigmoid(g[:, :, :ATOM])
    core = jax.nn.softplus(g[:, :, ATOM:])
    s = jnp.sum(filt * core, axis=1)
    s_ref[...] = s

    @pl.when(pl.program_id(0) == 0)
    def _():
        sum_ref[...] = jnp.zeros_like(sum_ref)
        sq_ref[...] = jnp.zeros_like(sq_ref)

    sum_ref[...] += jnp.sum(s, axis=0)[None, :]
    sq_ref[...] += jnp.sum(s * s, axis=0)[None, :]


def _gate(edges, nbr2, ys, We, scale1, shift1):
    return pl.pallas_call(
        _gate_body,
        grid=(EGRID,),
        in_specs=[
            pl.BlockSpec((EB, F2), lambda i: (i, 0)),
            pl.BlockSpec((EB, NBR), lambda i: (i, 0)),
            pl.BlockSpec((AB, F2), lambda i: (i, 0)),
            pl.BlockSpec((NBR, F2), lambda i: (0, 0)),
            pl.BlockSpec((1, F2), lambda i: (0, 0)),
            pl.BlockSpec((1, F2), lambda i: (0, 0)),
        ],
        out_specs=[
            pl.BlockSpec((AB, ATOM), lambda i: (i, 0)),
            pl.BlockSpec((1, ATOM), lambda i: (0, 0)),
            pl.BlockSpec((1, ATOM), lambda i: (0, 0)),
        ],
        out_shape=[
            jax.ShapeDtypeStruct((N, ATOM), jnp.float32),
            jax.ShapeDtypeStruct((1, ATOM), jnp.float32),
            jax.ShapeDtypeStruct((1, ATOM), jnp.float32),
        ],
    )(edges, nbr2, ys, We, scale1, shift1)


def _update_body(x_ref, s_ref, sc_ref, sh_ref, o_ref):
    o_ref[...] = jax.nn.softplus(
        x_ref[...] + s_ref[...] * sc_ref[...] + sh_ref[...])


def _update(x, s, scale2, shift2):
    RB = 2000
    return pl.pallas_call(
        _update_body,
        grid=(N // RB,),
        in_specs=[
            pl.BlockSpec((RB, ATOM), lambda i: (i, 0)),
            pl.BlockSpec((RB, ATOM), lambda i: (i, 0)),
            pl.BlockSpec((1, ATOM), lambda i: (0, 0)),
            pl.BlockSpec((1, ATOM), lambda i: (0, 0)),
        ],
        out_specs=pl.BlockSpec((RB, ATOM), lambda i: (i, 0)),
        out_shape=jax.ShapeDtypeStruct((N, ATOM), jnp.float32),
    )(x, s, scale2, shift2)


def _head_body(x_ref, cw_ref, cb_ref, ow_ref, ob_ref, o_ref):
    xm = jnp.mean(x_ref[...].reshape(N0, APC, ATOM), axis=1)
    a = jnp.dot(jax.nn.softplus(xm), cw_ref[...],
                preferred_element_type=jnp.float32,
                precision=lax.Precision.HIGHEST) + cb_ref[...]
    a = jax.nn.softplus(a)
    o_ref[...] = jnp.dot(a, ow_ref[...],
                         preferred_element_type=jnp.float32,
                         precision=lax.Precision.HIGHEST) + ob_ref[...]


def _head(x, ctf_W, ctf_b, out_W, out_b):
    return pl.pallas_call(
        _head_body,
        in_specs=[
            pl.BlockSpec((N, ATOM), lambda: (0, 0)),
            pl.BlockSpec((ATOM, H), lambda: (0, 0)),
            pl.BlockSpec((1, H), lambda: (0, 0)),
            pl.BlockSpec((H, 1), lambda: (0, 0)),
            pl.BlockSpec((1, 1), lambda: (0, 0)),
        ],
        out_specs=pl.BlockSpec((N0, 1), lambda: (0, 0)),
        out_shape=jax.ShapeDtypeStruct((N0, 1), jnp.float32),
    )(x, ctf_W, ctf_b, out_W, out_b)


def kernel(atom_fea, nbr_fea, nbr_fea_idx, crystal_atom_idx, params):
    mask = params['mask'].reshape(1, ORIG)
    emb_b = params['emb_b'].reshape(1, ATOM)
    am, x = _embed(atom_fea, mask, params['emb_W'], emb_b)

    idx_flat = nbr_fea_idx.reshape(E).astype(jnp.int32)
    nbr2 = nbr_fea.reshape(E, NBR)

    for p in params['convs']:
        W = p['W']
        Ws, Wn, We = W[:ATOM], W[ATOM:2 * ATOM], W[2 * ATOM:]
        b = p['b'].reshape(1, F2)
        ys, yn = _dense(x, Ws, Wn, b)
        edges = _sc_gather(yn, idx_flat)
        gsum, gsq = _stats(edges, nbr2, ys, We)
        mean = gsum / E
        var = gsq / E - mean * mean
        rsig = lax.rsqrt(var + EPS)
        scale1 = p['g1'].reshape(1, F2) * rsig
        shift1 = p['b1'].reshape(1, F2) - mean * scale1
        s, ssum, ssq = _gate(edges, nbr2, ys, We, scale1, shift1)
        mean2 = ssum / N
        var2 = ssq / N - mean2 * mean2
        rsig2 = lax.rsqrt(var2 + EPS)
        scale2 = p['g2'].reshape(1, ATOM) * rsig2
        shift2 = p['b2'].reshape(1, ATOM) - mean2 * scale2
        x = _update(x, s, scale2, shift2)

    out = _head(x, params['ctf_W'], params['ctf_b'].reshape(1, H),
                params['out_W'], params['out_b'].reshape(1, 1))
    return out, am


# speedup vs baseline: 2.9376x; 1.5483x over previous
import functools

import jax
import jax.numpy as jnp
from jax import lax
from jax.experimental import pallas as pl
from jax.experimental.pallas import tpu as pltpu
from jax.experimental.pallas import tpu_sc as plsc

N = 50000
M = 16
ORIG = 92
NBR = 16
ATOM = 64
F2 = 2 * ATOM
H = 128
N0 = 1000
APC = 50
E = N * M
EPS = 1e-5

NW = 32
EPW = E // NW
CHUNK = 128
NFULL = EPW // CHUNK
TAIL = EPW - NFULL * CHUNK
NBUF = 2


def _sc_gather_body(table_hbm, idx_hbm, out_hbm, idx_v, rows_v, *sems):
    gsems = sems[:NBUF]
    wsems = sems[NBUF:2 * NBUF]
    isems = sems[2 * NBUF:]
    wid = lax.axis_index("s") * 2 + lax.axis_index("c")
    base = wid * EPW

    def fire_idx(g, b, n=CHUNK):
        pltpu.async_copy(
            idx_hbm.at[pl.ds(base + g * CHUNK, n)],
            idx_v.at[b].at[pl.ds(0, n)], isems[b])

    def wait_idx(b, n=CHUNK):
        pltpu.make_async_copy(
            idx_hbm.at[pl.ds(base, n)],
            idx_v.at[b].at[pl.ds(0, n)], isems[b]).wait()

    def fire_gather(b, n=CHUNK):
        pltpu.async_copy(
            table_hbm.at[idx_v.at[b].at[pl.ds(0, n)]],
            rows_v.at[b].at[pl.ds(0, n)], gsems[b])

    def drain_gather(b, n=CHUNK):
        pltpu.make_async_copy(
            rows_v.at[b].at[pl.ds(0, n)],
            out_hbm.at[pl.ds(base, n)], gsems[b]).wait()

    def fire_write(g, b, n=CHUNK):
        pltpu.async_copy(
            rows_v.at[b].at[pl.ds(0, n)],
            out_hbm.at[pl.ds(base + g * CHUNK, n)], wsems[b])

    def wait_write(b, n=CHUNK):
        pltpu.make_async_copy(
            rows_v.at[b].at[pl.ds(0, n)],
            out_hbm.at[pl.ds(base, n)], wsems[b]).wait()

    fire_idx(0, 0)
    fire_idx(1, 1)
    wait_idx(0)
    fire_gather(0)
    wait_idx(1)
    fire_gather(1)

    def body(i, carry):
        for b in range(NBUF):
            g = i * NBUF + b
            drain_gather(b)

            @pl.when(g + NBUF < NFULL)
            def _():
                fire_idx(g + NBUF, b)

            fire_write(g, b)

            @pl.when(g + NBUF < NFULL)
            def _():
                wait_write(b)
                wait_idx(b)
                fire_gather(b)
        return carry

    lax.fori_loop(0, (NFULL - 1) // NBUF, body, None)
    drain_gather(0)
    fire_write(NFULL - 1, 0)
    wait_write(1)
    fire_idx(NFULL, 1, TAIL)
    wait_idx(1, TAIL)
    fire_gather(1, TAIL)
    drain_gather(1, TAIL)
    fire_write(NFULL, 1, TAIL)
    wait_write(1, TAIL)
    wait_write(0)


def _sc_gather(table, idx_flat):
    mesh = plsc.VectorSubcoreMesh(core_axis_name="c", subcore_axis_name="s")
    return pl.kernel(
        _sc_gather_body,
        out_type=jax.ShapeDtypeStruct((E, F2), jnp.float32),
        mesh=mesh,
        scratch_types=(
            [pltpu.VMEM((NBUF, CHUNK), jnp.int32),
             pltpu.VMEM((NBUF, CHUNK, F2), jnp.float32)]
            + [pltpu.SemaphoreType.DMA] * (3 * NBUF)
        ),
    )(table, idx_flat)


def _embed_body(af_ref, mask_ref, w_ref, b_ref, am_ref, x_ref):
    am = af_ref[...] * mask_ref[...]
    am_ref[...] = am
    x_ref[...] = jnp.dot(am, w_ref[...],
                         preferred_element_type=jnp.float32,
                         precision=lax.Precision.HIGHEST) + b_ref[...]


def _embed(atom_fea, mask, emb_W, emb_b):
    RB = 2000
    grid = N // RB
    return pl.pallas_call(
        _embed_body,
        grid=(grid,),
        in_specs=[
            pl.BlockSpec((RB, ORIG), lambda i: (i, 0)),
            pl.BlockSpec((1, ORIG), lambda i: (0, 0)),
            pl.BlockSpec((ORIG, ATOM), lambda i: (0, 0)),
            pl.BlockSpec((1, ATOM), lambda i: (0, 0)),
        ],
        out_specs=[
            pl.BlockSpec((RB, ORIG), lambda i: (i, 0)),
            pl.BlockSpec((RB, ATOM), lambda i: (i, 0)),
        ],
        out_shape=[
            jax.ShapeDtypeStruct((N, ORIG), jnp.float32),
            jax.ShapeDtypeStruct((N, ATOM), jnp.float32),
        ],
    )(atom_fea, mask, emb_W, emb_b)


def _dense_body(x_ref, ws_ref, wn_ref, b_ref, ys_ref, yn_ref):
    x = x_ref[...]
    ys_ref[...] = jnp.dot(x, ws_ref[...],
                          preferred_element_type=jnp.float32,
                          precision=lax.Precision.HIGHEST) + b_ref[...]
    yn_ref[...] = jnp.dot(x, wn_ref[...],
                          preferred_element_type=jnp.float32,
                          precision=lax.Precision.HIGHEST)


def _dense(x, Ws, Wn, b):
    RB = 2000
    return pl.pallas_call(
        _dense_body,
        grid=(N // RB,),
        in_specs=[
            pl.BlockSpec((RB, ATOM), lambda i: (i, 0)),
            pl.BlockSpec((ATOM, F2), lambda i: (0, 0)),
            pl.BlockSpec((ATOM, F2), lambda i: (0, 0)),
            pl.BlockSpec((1, F2), lambda i: (0, 0)),
        ],
        out_specs=[
            pl.BlockSpec((RB, F2), lambda i: (i, 0)),
            pl.BlockSpec((RB, F2), lambda i: (i, 0)),
        ],
        out_shape=[
            jax.ShapeDtypeStruct((N, F2), jnp.float32),
            jax.ShapeDtypeStruct((N, F2), jnp.float32),
        ],
    )(x, Ws, Wn, b)


AB = 400
EGRID = N // AB
LOG2E = 1.4426950408889634
LN2 = 0.6931471805599453

_DN = (((2,), (0,)), ((), ()))


def _stats_body(edges_ref, nbr_ref, ys_ref, wem_ref, sum_ref, sq_ref):
    nbr = nbr_ref[...].astype(jnp.bfloat16)
    v = lax.dot_general(nbr, wem_ref[...], dimension_numbers=_DN,
                        preferred_element_type=jnp.float32)
    g = edges_ref[...] + v + ys_ref[...][None, :, :]

    @pl.when(pl.program_id(0) == 0)
    def _():
        sum_ref[...] = jnp.zeros_like(sum_ref)
        sq_ref[...] = jnp.zeros_like(sq_ref)

    sum_ref[...] += jnp.sum(g, axis=(0, 1))[None, :]
    sq_ref[...] += jnp.sum(g * g, axis=(0, 1))[None, :]


def _stats(edges3, nbr5, ys, WeM):
    return pl.pallas_call(
        _stats_body,
        grid=(EGRID,),
        in_specs=[
            pl.BlockSpec((M, AB, F2), lambda i: (0, i, 0)),
            pl.BlockSpec((M, AB, NBR), lambda i: (0, i, 0)),
            pl.BlockSpec((AB, F2), lambda i: (i, 0)),
            pl.BlockSpec((NBR, F2), lambda i: (0, 0)),
        ],
        out_specs=[
            pl.BlockSpec((1, F2), lambda i: (0, 0)),
            pl.BlockSpec((1, F2), lambda i: (0, 0)),
        ],
        out_shape=[
            jax.ShapeDtypeStruct((1, F2), jnp.float32),
            jax.ShapeDtypeStruct((1, F2), jnp.float32),
        ],
    )(edges3, nbr5, ys, WeM)


def _gate_body(edges_ref, nbr_ref, ys_ref, wem_ref, sc_ref, sh_ref,
               s_ref, sum_ref, sq_ref):
    nbr = nbr_ref[...].astype(jnp.bfloat16)
    v = lax.dot_general(nbr, wem_ref[...], dimension_numbers=_DN,
                        preferred_element_type=jnp.float32)
    ysg = ys_ref[...] * sc_ref[...] + sh_ref[...]
    g = edges_ref[...] * sc_ref[...][None, :, :] + v + ysg[None, :, :]
    filt2 = jnp.tanh(g[:, :, :ATOM]) + 1.0
    c = g[:, :, ATOM:]
    t = jnp.exp2(jnp.abs(c) * (-LOG2E))
    core = jnp.maximum(c, 0.0) + LN2 * jnp.log2(1.0 + t)
    s = jnp.sum(filt2 * core, axis=0)
    s_ref[...] = s

    @pl.when(pl.program_id(0) == 0)
    def _():
        sum_ref[...] = jnp.zeros_like(sum_ref)
        sq_ref[...] = jnp.zeros_like(sq_ref)

    sum_ref[...] += jnp.sum(s, axis=0)[None, :]
    sq_ref[...] += jnp.sum(s * s, axis=0)[None, :]


def _gate(edges3, nbr5, ys, WeMg, scale1, shift1):
    return pl.pallas_call(
        _gate_body,
        grid=(EGRID,),
        in_specs=[
            pl.BlockSpec((M, AB, F2), lambda i: (0, i, 0)),
            pl.BlockSpec((M, AB, NBR), lambda i: (0, i, 0)),
            pl.BlockSpec((AB, F2), lambda i: (i, 0)),
            pl.BlockSpec((NBR, F2), lambda i: (0, 0)),
            pl.BlockSpec((1, F2), lambda i: (0, 0)),
            pl.BlockSpec((1, F2), lambda i: (0, 0)),
        ],
        out_specs=[
            pl.BlockSpec((AB, ATOM), lambda i: (i, 0)),
            pl.BlockSpec((1, ATOM), lambda i: (0, 0)),
            pl.BlockSpec((1, ATOM), lambda i: (0, 0)),
        ],
        out_shape=[
            jax.ShapeDtypeStruct((N, ATOM), jnp.float32),
            jax.ShapeDtypeStruct((1, ATOM), jnp.float32),
            jax.ShapeDtypeStruct((1, ATOM), jnp.float32),
        ],
    )(edges3, nbr5, ys, WeMg, scale1, shift1)


def _update_body(x_ref, s_ref, sc_ref, sh_ref, o_ref):
    o_ref[...] = jax.nn.softplus(
        x_ref[...] + s_ref[...] * sc_ref[...] + sh_ref[...])


def _update(x, s, scale2, shift2):
    RB = 2000
    return pl.pallas_call(
        _update_body,
        grid=(N // RB,),
        in_specs=[
            pl.BlockSpec((RB, ATOM), lambda i: (i, 0)),
            pl.BlockSpec((RB, ATOM), lambda i: (i, 0)),
            pl.BlockSpec((1, ATOM), lambda i: (0, 0)),
            pl.BlockSpec((1, ATOM), lambda i: (0, 0)),
        ],
        out_specs=pl.BlockSpec((RB, ATOM), lambda i: (i, 0)),
        out_shape=jax.ShapeDtypeStruct((N, ATOM), jnp.float32),
    )(x, s, scale2, shift2)


def _head_body(x_ref, cw_ref, cb_ref, ow_ref, ob_ref, o_ref):
    xm = jnp.mean(x_ref[...].reshape(N0, APC, ATOM), axis=1)
    a = jnp.dot(jax.nn.softplus(xm), cw_ref[...],
                preferred_element_type=jnp.float32,
                precision=lax.Precision.HIGHEST) + cb_ref[...]
    a = jax.nn.softplus(a)
    o_ref[...] = jnp.dot(a, ow_ref[...],
                         preferred_element_type=jnp.float32,
                         precision=lax.Precision.HIGHEST) + ob_ref[...]


def _head(x, ctf_W, ctf_b, out_W, out_b):
    return pl.pallas_call(
        _head_body,
        in_specs=[
            pl.BlockSpec((N, ATOM), lambda: (0, 0)),
            pl.BlockSpec((ATOM, H), lambda: (0, 0)),
            pl.BlockSpec((1, H), lambda: (0, 0)),
            pl.BlockSpec((H, 1), lambda: (0, 0)),
            pl.BlockSpec((1, 1), lambda: (0, 0)),
        ],
        out_specs=pl.BlockSpec((N0, 1), lambda: (0, 0)),
        out_shape=jax.ShapeDtypeStruct((N0, 1), jnp.float32),
    )(x, ctf_W, ctf_b, out_W, out_b)


def kernel(atom_fea, nbr_fea, nbr_fea_idx, crystal_atom_idx, params):
    mask = params['mask'].reshape(1, ORIG)
    emb_b = params['emb_b'].reshape(1, ATOM)
    am, x = _embed(atom_fea, mask, params['emb_W'], emb_b)

    idx_flat = jnp.transpose(nbr_fea_idx).reshape(E).astype(jnp.int32)
    nbr3m = jnp.transpose(nbr_fea, (1, 0, 2))
    halfm = jnp.concatenate([jnp.full((1, ATOM), 0.5, jnp.float32),
                             jnp.ones((1, ATOM), jnp.float32)], axis=1)

    for p in params['convs']:
        W = p['W']
        Ws, Wn, We = W[:ATOM], W[ATOM:2 * ATOM], W[2 * ATOM:]
        WeM = We.astype(jnp.bfloat16)
        b = p['b'].reshape(1, F2)
        ys, yn = _dense(x, Ws, Wn, b)
        edges = _sc_gather(yn, idx_flat)
        edges3 = edges.reshape(M, N, F2)
        gsum, gsq = _stats(edges3, nbr3m, ys, WeM)
        mean = gsum / E
        var = gsq / E - mean * mean
        rsig = lax.rsqrt(var + EPS)
        scale1 = p['g1'].reshape(1, F2) * rsig * halfm
        shift1 = (p['b1'].reshape(1, F2) - mean * p['g1'].reshape(1, F2)
                  * rsig) * halfm
        WeMg = (We * scale1).astype(jnp.bfloat16)
        s, ssum, ssq = _gate(edges3, nbr3m, ys, WeMg, scale1, shift1)
        mean2 = ssum / (2 * N)
        var2 = ssq / (4 * N) - mean2 * mean2
        rsig2 = lax.rsqrt(var2 + EPS) * 0.5
        scale2 = p['g2'].reshape(1, ATOM) * rsig2
        shift2 = p['b2'].reshape(1, ATOM) - 2 * mean2 * scale2
        x = _update(x, s, scale2, shift2)

    out = _head(x, params['ctf_W'], params['ctf_b'].reshape(1, H),
                params['out_W'], params['out_b'].reshape(1, 1))
    return out, am


# speedup vs baseline: 3.4374x; 1.1701x over previous
import functools

import jax
import jax.numpy as jnp
from jax import lax
from jax.experimental import pallas as pl
from jax.experimental.pallas import tpu as pltpu
from jax.experimental.pallas import tpu_sc as plsc

N = 50000
M = 16
ORIG = 92
NBR = 16
ATOM = 64
F2 = 2 * ATOM
H = 128
N0 = 1000
APC = 50
E = N * M
EPS = 1e-5

NW = 32
EPW = E // NW
CHUNK = 128
NFULL = EPW // CHUNK
TAIL = EPW - NFULL * CHUNK
NBUF = 3


def _sc_gather_body(table_hbm, idx_hbm, out_hbm, idx_v, rows_v, *sems):
    gsems = sems[:NBUF]
    wsems = sems[NBUF:2 * NBUF]
    isems = sems[2 * NBUF:]
    wid = lax.axis_index("s") * 2 + lax.axis_index("c")
    base = wid * EPW

    def fire_idx(g, b, n=CHUNK):
        pltpu.async_copy(
            idx_hbm.at[pl.ds(base + g * CHUNK, n)],
            idx_v.at[b].at[pl.ds(0, n)], isems[b])

    def wait_idx(b, n=CHUNK):
        pltpu.make_async_copy(
            idx_hbm.at[pl.ds(base, n)],
            idx_v.at[b].at[pl.ds(0, n)], isems[b]).wait()

    def fire_gather(b, n=CHUNK):
        pltpu.async_copy(
            table_hbm.at[idx_v.at[b].at[pl.ds(0, n)]],
            rows_v.at[b].at[pl.ds(0, n)], gsems[b])

    def drain_gather(b, n=CHUNK):
        pltpu.make_async_copy(
            rows_v.at[b].at[pl.ds(0, n)],
            out_hbm.at[pl.ds(base, n)], gsems[b]).wait()

    def fire_write(g, b, n=CHUNK):
        pltpu.async_copy(
            rows_v.at[b].at[pl.ds(0, n)],
            out_hbm.at[pl.ds(base + g * CHUNK, n)], wsems[b])

    def wait_write(b, n=CHUNK):
        pltpu.make_async_copy(
            rows_v.at[b].at[pl.ds(0, n)],
            out_hbm.at[pl.ds(base, n)], wsems[b]).wait()

    for b in range(NBUF):
        fire_idx(b, b)
    for b in range(NBUF):
        wait_idx(b)
        fire_gather(b)

    def body(i, carry):
        for b in range(NBUF):
            g = i * NBUF + b
            drain_gather(b)

            @pl.when(g + NBUF < NFULL)
            def _():
                fire_idx(g + NBUF, b)

            fire_write(g, b)

            @pl.when(g + NBUF < NFULL)
            def _():
                wait_write(b)
                wait_idx(b)
                fire_gather(b)
        return carry

    lax.fori_loop(0, NFULL // NBUF, body, None)
    wait_write(0)
    fire_idx(NFULL, 0, TAIL)
    wait_idx(0, TAIL)
    fire_gather(0, TAIL)
    drain_gather(0, TAIL)
    fire_write(NFULL, 0, TAIL)
    wait_write(0, TAIL)
    for b in range(1, NBUF):
        wait_write(b)


def _sc_gather(table, idx_flat):
    mesh = plsc.VectorSubcoreMesh(core_axis_name="c", subcore_axis_name="s")
    return pl.kernel(
        _sc_gather_body,
        out_type=jax.ShapeDtypeStruct((E, F2), jnp.float32),
        mesh=mesh,
        scratch_types=(
            [pltpu.VMEM((NBUF, CHUNK), jnp.int32),
             pltpu.VMEM((NBUF, CHUNK, F2), jnp.float32)]
            + [pltpu.SemaphoreType.DMA] * (3 * NBUF)
        ),
    )(table, idx_flat)


def _embed_body(af_ref, mask_ref, w_ref, b_ref, am_ref, x_ref):
    am = af_ref[...] * mask_ref[...]
    am_ref[...] = am
    x_ref[...] = jnp.dot(am, w_ref[...],
                         preferred_element_type=jnp.float32,
                         precision=lax.Precision.HIGHEST) + b_ref[...]


def _embed(atom_fea, mask, emb_W, emb_b):
    RB = 2000
    grid = N // RB
    return pl.pallas_call(
        _embed_body,
        grid=(grid,),
        in_specs=[
            pl.BlockSpec((RB, ORIG), lambda i: (i, 0)),
            pl.BlockSpec((1, ORIG), lambda i: (0, 0)),
            pl.BlockSpec((ORIG, ATOM), lambda i: (0, 0)),
            pl.BlockSpec((1, ATOM), lambda i: (0, 0)),
        ],
        out_specs=[
            pl.BlockSpec((RB, ORIG), lambda i: (i, 0)),
            pl.BlockSpec((RB, ATOM), lambda i: (i, 0)),
        ],
        out_shape=[
            jax.ShapeDtypeStruct((N, ORIG), jnp.float32),
            jax.ShapeDtypeStruct((N, ATOM), jnp.float32),
        ],
    )(atom_fea, mask, emb_W, emb_b)


def _dense_body(x_ref, ws_ref, wn_ref, b_ref, ys_ref, yn_ref):
    x = x_ref[...]
    ys_ref[...] = jnp.dot(x, ws_ref[...],
                          preferred_element_type=jnp.float32,
                          precision=lax.Precision.HIGHEST) + b_ref[...]
    yn_ref[...] = jnp.dot(x, wn_ref[...],
                          preferred_element_type=jnp.float32,
                          precision=lax.Precision.HIGHEST)


def _dense(x, Ws, Wn, b):
    RB = 2000
    return pl.pallas_call(
        _dense_body,
        grid=(N // RB,),
        in_specs=[
            pl.BlockSpec((RB, ATOM), lambda i: (i, 0)),
            pl.BlockSpec((ATOM, F2), lambda i: (0, 0)),
            pl.BlockSpec((ATOM, F2), lambda i: (0, 0)),
            pl.BlockSpec((1, F2), lambda i: (0, 0)),
        ],
        out_specs=[
            pl.BlockSpec((RB, F2), lambda i: (i, 0)),
            pl.BlockSpec((RB, F2), lambda i: (i, 0)),
        ],
        out_shape=[
            jax.ShapeDtypeStruct((N, F2), jnp.float32),
            jax.ShapeDtypeStruct((N, F2), jnp.float32),
        ],
    )(x, Ws, Wn, b)


AB = 1000
EGRID = N // AB
LOG2E = 1.4426950408889634
LN2 = 0.6931471805599453

_DN = (((2,), (0,)), ((), ()))


def _stats_body(edges_ref, nbr_ref, ys_ref, wem_ref, sum_ref, sq_ref):
    v = lax.dot_general(nbr_ref[...], wem_ref[...], dimension_numbers=_DN,
                        preferred_element_type=jnp.float32)
    g = edges_ref[...] + v + ys_ref[...][None, :, :]

    @pl.when(pl.program_id(0) == 0)
    def _():
        sum_ref[...] = jnp.zeros_like(sum_ref)
        sq_ref[...] = jnp.zeros_like(sq_ref)

    sum_ref[...] += jnp.sum(g, axis=(0, 1))[None, :]
    sq_ref[...] += jnp.sum(g * g, axis=(0, 1))[None, :]


def _stats(edges3, nbr5, ys, WeM):
    return pl.pallas_call(
        _stats_body,
        grid=(EGRID,),
        in_specs=[
            pl.BlockSpec((M, AB, F2), lambda i: (0, i, 0)),
            pl.BlockSpec((M, AB, NBR), lambda i: (0, i, 0)),
            pl.BlockSpec((AB, F2), lambda i: (i, 0)),
            pl.BlockSpec((NBR, F2), lambda i: (0, 0)),
        ],
        out_specs=[
            pl.BlockSpec((1, F2), lambda i: (0, 0)),
            pl.BlockSpec((1, F2), lambda i: (0, 0)),
        ],
        out_shape=[
            jax.ShapeDtypeStruct((1, F2), jnp.float32),
            jax.ShapeDtypeStruct((1, F2), jnp.float32),
        ],
    )(edges3, nbr5, ys, WeM)


def _gate_body(edges_ref, nbr_ref, ys_ref, wem_ref, sc_ref, sh_ref,
               s_ref, sum_ref, sq_ref):
    v = lax.dot_general(nbr_ref[...], wem_ref[...], dimension_numbers=_DN,
                        preferred_element_type=jnp.float32)
    ysg = ys_ref[...] * sc_ref[...] + sh_ref[...]
    g = edges_ref[...] * sc_ref[...][None, :, :] + v + ysg[None, :, :]
    filt2 = jnp.tanh(g[:, :, :ATOM]) + 1.0
    c = g[:, :, ATOM:]
    t = jnp.exp2(jnp.abs(c) * (-LOG2E))
    core = jnp.maximum(c, 0.0) + LN2 * jnp.log2(1.0 + t)
    s = jnp.sum(filt2 * core, axis=0)
    s_ref[...] = s

    @pl.when(pl.program_id(0) == 0)
    def _():
        sum_ref[...] = jnp.zeros_like(sum_ref)
        sq_ref[...] = jnp.zeros_like(sq_ref)

    sum_ref[...] += jnp.sum(s, axis=0)[None, :]
    sq_ref[...] += jnp.sum(s * s, axis=0)[None, :]


def _gate(edges3, nbr5, ys, WeMg, scale1, shift1):
    return pl.pallas_call(
        _gate_body,
        grid=(EGRID,),
        in_specs=[
            pl.BlockSpec((M, AB, F2), lambda i: (0, i, 0)),
            pl.BlockSpec((M, AB, NBR), lambda i: (0, i, 0)),
            pl.BlockSpec((AB, F2), lambda i: (i, 0)),
            pl.BlockSpec((NBR, F2), lambda i: (0, 0)),
            pl.BlockSpec((1, F2), lambda i: (0, 0)),
            pl.BlockSpec((1, F2), lambda i: (0, 0)),
        ],
        out_specs=[
            pl.BlockSpec((AB, ATOM), lambda i: (i, 0)),
            pl.BlockSpec((1, ATOM), lambda i: (0, 0)),
            pl.BlockSpec((1, ATOM), lambda i: (0, 0)),
        ],
        out_shape=[
            jax.ShapeDtypeStruct((N, ATOM), jnp.float32),
            jax.ShapeDtypeStruct((1, ATOM), jnp.float32),
            jax.ShapeDtypeStruct((1, ATOM), jnp.float32),
        ],
    )(edges3, nbr5, ys, WeMg, scale1, shift1)


def _update_body(x_ref, s_ref, sc_ref, sh_ref, o_ref):
    o_ref[...] = jax.nn.softplus(
        x_ref[...] + s_ref[...] * sc_ref[...] + sh_ref[...])


def _update(x, s, scale2, shift2):
    RB = 2000
    return pl.pallas_call(
        _update_body,
        grid=(N // RB,),
        in_specs=[
            pl.BlockSpec((RB, ATOM), lambda i: (i, 0)),
            pl.BlockSpec((RB, ATOM), lambda i: (i, 0)),
            pl.BlockSpec((1, ATOM), lambda i: (0, 0)),
            pl.BlockSpec((1, ATOM), lambda i: (0, 0)),
        ],
        out_specs=pl.BlockSpec((RB, ATOM), lambda i: (i, 0)),
        out_shape=jax.ShapeDtypeStruct((N, ATOM), jnp.float32),
    )(x, s, scale2, shift2)


def _head_body(x_ref, cw_ref, cb_ref, ow_ref, ob_ref, o_ref):
    xm = jnp.mean(x_ref[...].reshape(N0, APC, ATOM), axis=1)
    a = jnp.dot(jax.nn.softplus(xm), cw_ref[...],
                preferred_element_type=jnp.float32,
                precision=lax.Precision.HIGHEST) + cb_ref[...]
    a = jax.nn.softplus(a)
    o_ref[...] = jnp.dot(a, ow_ref[...],
                         preferred_element_type=jnp.float32,
                         precision=lax.Precision.HIGHEST) + ob_ref[...]


def _head(x, ctf_W, ctf_b, out_W, out_b):
    return pl.pallas_call(
        _head_body,
        in_specs=[
            pl.BlockSpec((N, ATOM), lambda: (0, 0)),
            pl.BlockSpec((ATOM, H), lambda: (0, 0)),
            pl.BlockSpec((1, H), lambda: (0, 0)),
            pl.BlockSpec((H, 1), lambda: (0, 0)),
            pl.BlockSpec((1, 1), lambda: (0, 0)),
        ],
        out_specs=pl.BlockSpec((N0, 1), lambda: (0, 0)),
        out_shape=jax.ShapeDtypeStruct((N0, 1), jnp.float32),
    )(x, ctf_W, ctf_b, out_W, out_b)


def kernel(atom_fea, nbr_fea, nbr_fea_idx, crystal_atom_idx, params):
    mask = params['mask'].reshape(1, ORIG)
    emb_b = params['emb_b'].reshape(1, ATOM)
    am, x = _embed(atom_fea, mask, params['emb_W'], emb_b)

    idx_flat = jnp.transpose(nbr_fea_idx).reshape(E).astype(jnp.int32)
    nbr3m = jnp.transpose(nbr_fea, (1, 0, 2)).astype(jnp.bfloat16)
    halfm = jnp.concatenate([jnp.full((1, ATOM), 0.5, jnp.float32),
                             jnp.ones((1, ATOM), jnp.float32)], axis=1)

    for p in params['convs']:
        W = p['W']
        Ws, Wn, We = W[:ATOM], W[ATOM:2 * ATOM], W[2 * ATOM:]
        WeM = We.astype(jnp.bfloat16)
        b = p['b'].reshape(1, F2)
        ys, yn = _dense(x, Ws, Wn, b)
        edges = _sc_gather(yn, idx_flat)
        edges3 = edges.reshape(M, N, F2)
        gsum, gsq = _stats(edges3, nbr3m, ys, WeM)
        mean = gsum / E
        var = gsq / E - mean * mean
        rsig = lax.rsqrt(var + EPS)
        scale1 = p['g1'].reshape(1, F2) * rsig * halfm
        shift1 = (p['b1'].reshape(1, F2) - mean * p['g1'].reshape(1, F2)
                  * rsig) * halfm
        WeMg = (We * scale1).astype(jnp.bfloat16)
        s, ssum, ssq = _gate(edges3, nbr3m, ys, WeMg, scale1, shift1)
        mean2 = ssum / (2 * N)
        var2 = ssq / (4 * N) - mean2 * mean2
        rsig2 = lax.rsqrt(var2 + EPS) * 0.5
        scale2 = p['g2'].reshape(1, ATOM) * rsig2
        shift2 = p['b2'].reshape(1, ATOM) - 2 * mean2 * scale2
        x = _update(x, s, scale2, shift2)

    out = _head(x, params['ctf_W'], params['ctf_b'].reshape(1, H),
                params['out_W'], params['out_b'].reshape(1, 1))
    return out, am
